# baseline (speedup 1.0000x reference)
import functools

import jax
import jax.numpy as jnp
from jax import lax
from jax.experimental import pallas as pl
from jax.experimental.pallas import tpu as pltpu
import jax.experimental.pallas.tpu_sc as plsc

N_NODES = 10000
NC, NS = 2, 16
NW = NC * NS
CHUNK = 88
NPAD = 10112
RPT = NPAD // NS
F = 128


@functools.cache
def _mesh():
    return plsc.VectorSubcoreMesh(
        core_axis_name="c", subcore_axis_name="s", num_cores=NC, num_subcores=NS
    )


def _make_agg(m0, m1):
    @functools.partial(
        pl.kernel,
        out_type=jax.ShapeDtypeStruct((NC, NPAD, F), jnp.float32),
        mesh=_mesh(),
        scratch_types=[
            pltpu.VMEM((CHUNK,), jnp.int32),
            pltpu.VMEM((CHUNK,), jnp.int32),
            pltpu.VMEM((CHUNK,), jnp.int32),
            pltpu.VMEM((CHUNK,), jnp.int32),
            pltpu.VMEM((CHUNK,), jnp.int32),
            pltpu.VMEM((CHUNK,), jnp.int32),
            pltpu.VMEM((CHUNK,), jnp.int32),
            pltpu.VMEM((CHUNK,), jnp.int32),
            pltpu.VMEM((CHUNK,), jnp.int32),
            pltpu.VMEM((CHUNK,), jnp.int32),
            pltpu.VMEM((4, CHUNK, F), jnp.float32),
            pltpu.VMEM_SHARED((NPAD, F), jnp.float32),
        ]
        + [pltpu.SemaphoreType.DMA] * 18,
    )
    def agg(hp_hbm, src_hbm, dst_hbm, out_hbm,
            s0, s1, s2, s3, d0, d1, d2, d3, d4, d5, rows, acc, *sems):
        sbuf = [s0, s1, s2, s3]
        dbuf = [d0, d1, d2, d3, d4, d5]
        issem = sems[0:4]
        idsem = sems[4:10]
        gsem = sems[10:14]
        ssem = sems[14:18]
        cid = lax.axis_index("c")
        sid = lax.axis_index("s")
        row0 = sid * RPT

        def pipeline(n_chunks, c0):
            def isrc_load(j):
                return pltpu.async_copy(
                    src_hbm.at[pl.ds(c0 + j * CHUNK, CHUNK)], sbuf[j % 4], issem[j % 4]
                )

            def idst_load(j):
                return pltpu.async_copy(
                    dst_hbm.at[pl.ds(c0 + j * CHUNK, CHUNK)], dbuf[j % 6], idsem[j % 6]
                )

            def gather(j):
                return pltpu.async_copy(
                    hp_hbm.at[sbuf[j % 4]], rows.at[j % 4], gsem[j % 4]
                )

            def scatter(j):
                return pltpu.async_copy(
                    rows.at[j % 4], acc.at[dbuf[j % 6]], ssem[j % 4], add=True
                )

            isrc = {j: isrc_load(j) for j in range(min(4, n_chunks))}
            idst = {j: idst_load(j) for j in range(min(4, n_chunks))}
            gdesc = {}
            for j in range(min(2, n_chunks)):
                isrc.pop(j).wait()
                gdesc[j] = gather(j)
            sdesc = {}
            for j in range(n_chunks):
                gdesc.pop(j).wait()
                idst.pop(j).wait()
                sdesc[j] = scatter(j)
                if j >= 2:
                    sdesc.pop(j - 2).wait()
                if j + 2 < n_chunks:
                    isrc.pop(j + 2).wait()
                    gdesc[j + 2] = gather(j + 2)
                if j + 4 < n_chunks:
                    isrc[j + 4] = isrc_load(j + 4)
                    idst[j + 4] = idst_load(j + 4)
            for j in range(max(0, n_chunks - 2), n_chunks):
                sdesc.pop(j).wait()

        pltpu.sync_copy(hp_hbm.at[pl.ds(row0, RPT)], acc.at[pl.ds(row0, RPT)])
        plsc.subcore_barrier()

        @pl.when(cid == 0)
        def _():
            pipeline(m0, sid * (m0 * CHUNK))

        @pl.when(cid == 1)
        def _():
            pipeline(m1, (NS * m0 + sid * m1) * CHUNK)

        plsc.subcore_barrier()
        pltpu.sync_copy(acc.at[pl.ds(row0, RPT)], out_hbm.at[cid, pl.ds(row0, RPT)])

    return agg


def _make_deg(e_pad):
    ept = e_pad // NW
    n_chunks = ept // CHUNK

    @functools.partial(
        pl.kernel,
        out_type=jax.ShapeDtypeStruct((NC, NPAD, F), jnp.float32),
        mesh=_mesh(),
        scratch_types=[
            pltpu.VMEM((CHUNK,), jnp.int32),
            pltpu.VMEM((CHUNK,), jnp.int32),
            pltpu.VMEM((CHUNK,), jnp.int32),
            pltpu.VMEM((CHUNK,), jnp.int32),
            pltpu.VMEM((CHUNK, F), jnp.float32),
            pltpu.VMEM_SHARED((NPAD, F), jnp.float32),
        ]
        + [pltpu.SemaphoreType.DMA] * 6,
    )
    def deg(ones_hbm, dst_hbm, out_hbm, d0, d1, d2, d3, ones_v, acc, *sems):
        dbuf = [d0, d1, d2, d3]
        idsem = sems[0:4]
        ssem = sems[4:6]
        cid = lax.axis_index("c")
        sid = lax.axis_index("s")
        wid = cid * NS + sid
        row0 = sid * RPT
        c0 = wid * ept

        def idst_load(j):
            return pltpu.async_copy(
                dst_hbm.at[pl.ds(c0 + j * CHUNK, CHUNK)], dbuf[j % 4], idsem[j % 4]
            )

        idst = {j: idst_load(j) for j in range(min(3, n_chunks))}
        pltpu.sync_copy(ones_hbm.at[pl.ds(0, CHUNK)], ones_v)
        pltpu.sync_copy(ones_hbm.at[pl.ds(row0, RPT)], acc.at[pl.ds(row0, RPT)])
        plsc.subcore_barrier()
        sdesc = {}
        for j in range(n_chunks):
            idst.pop(j).wait()
            sdesc[j] = pltpu.async_copy(ones_v, acc.at[dbuf[j % 4]], ssem[j % 2], add=True)
            if j >= 1:
                sdesc.pop(j - 1).wait()
            if j + 3 < n_chunks:
                idst[j + 3] = idst_load(j + 3)
        sdesc.pop(n_chunks - 1).wait()
        plsc.subcore_barrier()
        pltpu.sync_copy(acc.at[pl.ds(row0, RPT)], out_hbm.at[cid, pl.ds(row0, RPT)])

    return deg


def _tc_pre(degp_ref, x_ref, w_ref, dinv_ref, h1_ref):
    deg = degp_ref[0] + degp_ref[1] - 1.0
    dinv = lax.rsqrt(deg)
    dinv_ref[...] = dinv
    h1_ref[...] = jnp.dot(x_ref[...], w_ref[...], preferred_element_type=jnp.float32) * dinv


def _make_tc_mid(relu, emit_a):
    def body(s_ref, hp_ref, dinv_ref, b_ref, w_ref, *outs):
        dinv = dinv_ref[...]
        a = dinv * (s_ref[0] + s_ref[1] - hp_ref[...]) + b_ref[...]
        if relu:
            a = jnp.maximum(a, 0.0)
        if emit_a:
            outs[0][...] = a
        outs[-1][...] = jnp.dot(a, w_ref[...], preferred_element_type=jnp.float32) * dinv

    return body


def _tc_final(s_ref, hp_ref, dinv_ref, b_ref, xr_ref):
    xr_ref[...] = dinv_ref[...] * (s_ref[0] + s_ref[1] - hp_ref[...]) + b_ref[...]


def _sds(shape):
    return jax.ShapeDtypeStruct(shape, jnp.float32)


def _pad_w(w):
    return jnp.pad(w, ((0, F - w.shape[0]), (0, F - w.shape[1])))


def _pad_b(b):
    return jnp.pad(b, (0, F - b.shape[0])).reshape(1, F)


def kernel(x, edge_index, W1, b1, W2, b2, W3, b3, W4, b4):
    n, d_in = x.shape
    e = edge_index.shape[1]
    d_l = W2.shape[1]

    mtot = -(-e // (NS * CHUNK))
    mtot += mtot % 2
    m0 = round(mtot * 5 / 12)
    m1 = mtot - m0
    e_pad = NS * mtot * CHUNK

    src = edge_index[0].astype(jnp.int32)
    dst = edge_index[1].astype(jnp.int32)
    src = jnp.concatenate([src, jnp.zeros((e_pad - e,), jnp.int32)])
    dst = jnp.concatenate([dst, jnp.full((e_pad - e,), n, jnp.int32)])
    xp = jnp.pad(x, ((0, NPAD - n), (0, F - d_in)))
    ones = jnp.ones((NPAD, F), jnp.float32)

    w1, w2, w3, w4 = _pad_w(W1), _pad_w(W2), _pad_w(W3), _pad_w(W4)
    c1, c2, c3, c4 = _pad_b(b1), _pad_b(b2), _pad_b(b3), _pad_b(b4)

    agg = _make_agg(m0, m1)

    degp = _make_deg(e_pad)(ones, dst)
    dinv, h1 = pl.pallas_call(
        _tc_pre, out_shape=[_sds((NPAD, F)), _sds((NPAD, F))]
    )(degp, xp, w1)
    s1 = agg(h1, src, dst)
    (h2,) = pl.pallas_call(
        _make_tc_mid(relu=True, emit_a=False), out_shape=[_sds((NPAD, F))]
    )(s1, h1, dinv, c1, w2)
    s2 = agg(h2, src, dst)
    z, h3 = pl.pallas_call(
        _make_tc_mid(relu=False, emit_a=True), out_shape=[_sds((NPAD, F)), _sds((NPAD, F))]
    )(s2, h2, dinv, c2, w3)
    s3 = agg(h3, src, dst)
    (h4,) = pl.pallas_call(
        _make_tc_mid(relu=True, emit_a=False), out_shape=[_sds((NPAD, F))]
    )(s3, h3, dinv, c3, w4)
    s4 = agg(h4, src, dst)
    xr = pl.pallas_call(
        _tc_final, out_shape=_sds((NPAD, F))
    )(s4, h4, dinv, c4)
    return xr[:n, :d_in], z[:n, :d_l]


# speedup vs baseline: 1.1166x; 1.1166x over previous
import functools

import jax
import jax.numpy as jnp
from jax import lax
from jax.experimental import pallas as pl
from jax.experimental.pallas import tpu as pltpu
import jax.experimental.pallas.tpu_sc as plsc

N_NODES = 10000
NC, NS = 2, 16
NW = NC * NS
CHUNK = 88
NPAD = 10112
RPT = NPAD // NS
F = 128


@functools.cache
def _mesh():
    return plsc.VectorSubcoreMesh(
        core_axis_name="c", subcore_axis_name="s", num_cores=NC, num_subcores=NS
    )


def _make_agg(m0, m1):
    @functools.partial(
        pl.kernel,
        out_type=jax.ShapeDtypeStruct((NC, NPAD, F), jnp.float32),
        mesh=_mesh(),
        scratch_types=[
            pltpu.VMEM((CHUNK,), jnp.int32),
            pltpu.VMEM((CHUNK,), jnp.int32),
            pltpu.VMEM((CHUNK,), jnp.int32),
            pltpu.VMEM((CHUNK,), jnp.int32),
            pltpu.VMEM((CHUNK,), jnp.int32),
            pltpu.VMEM((CHUNK,), jnp.int32),
            pltpu.VMEM((CHUNK,), jnp.int32),
            pltpu.VMEM((CHUNK,), jnp.int32),
            pltpu.VMEM((CHUNK,), jnp.int32),
            pltpu.VMEM((CHUNK,), jnp.int32),
            pltpu.VMEM((4, CHUNK, F), jnp.float32),
            pltpu.VMEM_SHARED((NPAD, F), jnp.float32),
        ]
        + [pltpu.SemaphoreType.DMA] * 18,
    )
    def agg(hp_hbm, src_hbm, dst_hbm, out_hbm,
            s0, s1, s2, s3, d0, d1, d2, d3, d4, d5, rows, acc, *sems):
        sbuf = [s0, s1, s2, s3]
        dbuf = [d0, d1, d2, d3, d4, d5]
        issem = sems[0:4]
        idsem = sems[4:10]
        gsem = sems[10:14]
        ssem = sems[14:18]
        cid = lax.axis_index("c")
        sid = lax.axis_index("s")
        row0 = sid * RPT

        def pipeline(n_chunks, c0):
            def isrc_load(j):
                return pltpu.async_copy(
                    src_hbm.at[pl.ds(c0 + j * CHUNK, CHUNK)], sbuf[j % 4], issem[j % 4]
                )

            def idst_load(j):
                return pltpu.async_copy(
                    dst_hbm.at[pl.ds(c0 + j * CHUNK, CHUNK)], dbuf[j % 6], idsem[j % 6]
                )

            def gather(j):
                return pltpu.async_copy(
                    hp_hbm.at[sbuf[j % 4]], rows.at[j % 4], gsem[j % 4]
                )

            def scatter(j):
                return pltpu.async_copy(
                    rows.at[j % 4], acc.at[dbuf[j % 6]], ssem[j % 4], add=True
                )

            isrc = {j: isrc_load(j) for j in range(min(4, n_chunks))}
            idst = {j: idst_load(j) for j in range(min(4, n_chunks))}
            gdesc = {}
            for j in range(min(2, n_chunks)):
                isrc.pop(j).wait()
                gdesc[j] = gather(j)
            sdesc = {}
            for j in range(n_chunks):
                gdesc.pop(j).wait()
                idst.pop(j).wait()
                sdesc[j] = scatter(j)
                if j >= 2:
                    sdesc.pop(j - 2).wait()
                if j + 2 < n_chunks:
                    isrc.pop(j + 2).wait()
                    gdesc[j + 2] = gather(j + 2)
                if j + 4 < n_chunks:
                    isrc[j + 4] = isrc_load(j + 4)
                    idst[j + 4] = idst_load(j + 4)
            for j in range(max(0, n_chunks - 2), n_chunks):
                sdesc.pop(j).wait()

        pltpu.sync_copy(hp_hbm.at[pl.ds(row0, RPT)], acc.at[pl.ds(row0, RPT)])
        plsc.subcore_barrier()

        @pl.when(cid == 0)
        def _():
            pipeline(m0, sid * (m0 * CHUNK))

        @pl.when(cid == 1)
        def _():
            pipeline(m1, (NS * m0 + sid * m1) * CHUNK)

        plsc.subcore_barrier()
        pltpu.sync_copy(acc.at[pl.ds(row0, RPT)], out_hbm.at[cid, pl.ds(row0, RPT)])

    return agg


def _make_deg(e_pad):
    ept = e_pad // NW
    n_chunks = ept // CHUNK

    @functools.partial(
        pl.kernel,
        out_type=jax.ShapeDtypeStruct((NC, NPAD, F), jnp.float32),
        mesh=_mesh(),
        scratch_types=[
            pltpu.VMEM((CHUNK,), jnp.int32),
            pltpu.VMEM((CHUNK,), jnp.int32),
            pltpu.VMEM((CHUNK,), jnp.int32),
            pltpu.VMEM((CHUNK,), jnp.int32),
            pltpu.VMEM((CHUNK, F), jnp.float32),
            pltpu.VMEM_SHARED((NPAD, F), jnp.float32),
        ]
        + [pltpu.SemaphoreType.DMA] * 6,
    )
    def deg(ones_hbm, dst_hbm, out_hbm, d0, d1, d2, d3, ones_v, acc, *sems):
        dbuf = [d0, d1, d2, d3]
        idsem = sems[0:4]
        ssem = sems[4:6]
        cid = lax.axis_index("c")
        sid = lax.axis_index("s")
        wid = cid * NS + sid
        row0 = sid * RPT
        c0 = wid * ept

        def idst_load(j):
            return pltpu.async_copy(
                dst_hbm.at[pl.ds(c0 + j * CHUNK, CHUNK)], dbuf[j % 4], idsem[j % 4]
            )

        idst = {j: idst_load(j) for j in range(min(3, n_chunks))}
        pltpu.sync_copy(ones_hbm.at[pl.ds(0, CHUNK)], ones_v)
        pltpu.sync_copy(ones_hbm.at[pl.ds(row0, RPT)], acc.at[pl.ds(row0, RPT)])
        plsc.subcore_barrier()
        sdesc = {}
        for j in range(n_chunks):
            idst.pop(j).wait()
            sdesc[j] = pltpu.async_copy(ones_v, acc.at[dbuf[j % 4]], ssem[j % 2], add=True)
            if j >= 1:
                sdesc.pop(j - 1).wait()
            if j + 3 < n_chunks:
                idst[j + 3] = idst_load(j + 3)
        sdesc.pop(n_chunks - 1).wait()
        plsc.subcore_barrier()
        pltpu.sync_copy(acc.at[pl.ds(row0, RPT)], out_hbm.at[cid, pl.ds(row0, RPT)])

    return deg


def _tc_pre(degp_ref, x_ref, w_ref, dinv_ref, h1_ref):
    deg = degp_ref[0] + degp_ref[1] - 1.0
    dinv = lax.rsqrt(deg)
    dinv_ref[...] = dinv
    h1_ref[...] = jnp.dot(x_ref[...], w_ref[...], preferred_element_type=jnp.float32) * dinv


def _make_tc_mid(relu, emit_a):
    def body(s_ref, hp_ref, dinv_ref, b_ref, w_ref, *outs):
        dinv = dinv_ref[...]
        a = dinv * (s_ref[0] + s_ref[1] - hp_ref[...]) + b_ref[...]
        if relu:
            a = jnp.maximum(a, 0.0)
        if emit_a:
            outs[0][...] = a
        outs[-1][...] = jnp.dot(a, w_ref[...], preferred_element_type=jnp.float32) * dinv

    return body


def _tc_final(s_ref, hp_ref, dinv_ref, b_ref, xr_ref):
    xr_ref[...] = dinv_ref[...] * (s_ref[0] + s_ref[1] - hp_ref[...]) + b_ref[...]


def _sds(shape):
    return jax.ShapeDtypeStruct(shape, jnp.float32)


def _pad_w(w):
    return jnp.pad(w, ((0, F - w.shape[0]), (0, F - w.shape[1])))


def _pad_b(b):
    return jnp.pad(b, (0, F - b.shape[0])).reshape(1, F)


def kernel(x, edge_index, W1, b1, W2, b2, W3, b3, W4, b4):
    n, d_in = x.shape
    e = edge_index.shape[1]
    d_l = W2.shape[1]

    mtot = -(-e // (NS * CHUNK))
    mtot += mtot % 2
    m0 = round(mtot * 7 / 12)
    m1 = mtot - m0
    e_pad = NS * mtot * CHUNK

    src = edge_index[0].astype(jnp.int32)
    dst = edge_index[1].astype(jnp.int32)
    src = jnp.concatenate([src, jnp.zeros((e_pad - e,), jnp.int32)])
    dst = jnp.concatenate([dst, jnp.full((e_pad - e,), n, jnp.int32)])
    xp = jnp.pad(x, ((0, NPAD - n), (0, F - d_in)))
    ones = jnp.ones((NPAD, F), jnp.float32)

    w1, w2, w3, w4 = _pad_w(W1), _pad_w(W2), _pad_w(W3), _pad_w(W4)
    c1, c2, c3, c4 = _pad_b(b1), _pad_b(b2), _pad_b(b3), _pad_b(b4)

    agg = _make_agg(m0, m1)

    degp = _make_deg(e_pad)(ones, dst)
    dinv, h1 = pl.pallas_call(
        _tc_pre, out_shape=[_sds((NPAD, F)), _sds((NPAD, F))]
    )(degp, xp, w1)
    s1 = agg(h1, src, dst)
    (h2,) = pl.pallas_call(
        _make_tc_mid(relu=True, emit_a=False), out_shape=[_sds((NPAD, F))]
    )(s1, h1, dinv, c1, w2)
    s2 = agg(h2, src, dst)
    z, h3 = pl.pallas_call(
        _make_tc_mid(relu=False, emit_a=True), out_shape=[_sds((NPAD, F)), _sds((NPAD, F))]
    )(s2, h2, dinv, c2, w3)
    s3 = agg(h3, src, dst)
    (h4,) = pl.pallas_call(
        _make_tc_mid(relu=True, emit_a=False), out_shape=[_sds((NPAD, F))]
    )(s3, h3, dinv, c3, w4)
    s4 = agg(h4, src, dst)
    xr = pl.pallas_call(
        _tc_final, out_shape=_sds((NPAD, F))
    )(s4, h4, dinv, c4)
    return xr[:n, :d_in], z[:n, :d_l]
